# initial kernel scaffold (unmeasured)
import jax
import jax.numpy as jnp
from jax import lax
from jax.experimental import pallas as pl
from jax.experimental.pallas import tpu as pltpu

N_DEV = 8
SQ = 512
D = 1024
HQ = 8
DH = 128
SKV = 2048
SCALE = 0.08838834764831843
BF16 = jnp.bfloat16
F32 = jnp.float32


def _attend(xb, wq_ref, wo_ref, k_ref, v_ref):
    q = jnp.dot(xb, wq_ref[...], preferred_element_type=F32).astype(BF16)

    def head(h, attn):
        qh = lax.dynamic_slice_in_dim(q, h * DH, DH, axis=1)
        s = lax.dot_general(qh, k_ref[h], (((1,), (1,)), ((), ())),
                            preferred_element_type=F32)
        m = jnp.max(s, axis=1, keepdims=True)
        p = jnp.exp(s - m)
        l = jnp.sum(p, axis=1, keepdims=True)
        o = jnp.dot(p.astype(BF16), v_ref[h], preferred_element_type=F32)
        o = (o / l).astype(BF16)
        return lax.dynamic_update_slice_in_dim(attn, o, h * DH, axis=1)

    attn = lax.fori_loop(0, HQ, head, jnp.zeros((SQ, HQ * DH), BF16))
    return jnp.dot(attn, wo_ref[...], preferred_element_type=F32)


def _body(x_ref, wq_ref, wo_ref, k_ref, v_ref, out_ref,
          xfull, partial, rs_in, rs_out,
          ag_send, ag_recv, rs_send, rs_recv):
    j = lax.axis_index("i")
    right = lax.rem(j + 1, N_DEV)
    left = lax.rem(j + N_DEV - 1, N_DEV)

    barrier = pltpu.get_barrier_semaphore()
    for nbr in (left, right):
        pl.semaphore_signal(barrier, inc=1, device_id=(nbr,),
                            device_id_type=pl.DeviceIdType.MESH)
    pl.semaphore_wait(barrier, 2)

    for t in range(N_DEV - 1):
        src = x_ref if t == 0 else xfull.at[t - 1]
        rdma = pltpu.make_async_remote_copy(
            src_ref=src,
            dst_ref=xfull.at[t],
            send_sem=ag_send.at[t],
            recv_sem=ag_recv.at[t],
            device_id=(right,),
            device_id_type=pl.DeviceIdType.MESH,
        )
        rdma.start()
        xb = x_ref[...] if t == 0 else xfull[t - 1]
        partial[t, :, :] = _attend(xb, wq_ref, wo_ref, k_ref, v_ref).astype(BF16)
        rdma.wait()
    partial[N_DEV - 1, :, :] = _attend(
        xfull[N_DEV - 2], wq_ref, wo_ref, k_ref, v_ref).astype(BF16)

    rs_out[...] = partial[1]
    for r in range(N_DEV - 1):
        rdma = pltpu.make_async_remote_copy(
            src_ref=rs_out,
            dst_ref=rs_in.at[r],
            send_sem=rs_send.at[r],
            recv_sem=rs_recv.at[r],
            device_id=(right,),
            device_id_type=pl.DeviceIdType.MESH,
        )
        rdma.start()
        rdma.wait()
        if r < N_DEV - 2:
            rs_out[...] = (rs_in[r].astype(F32)
                           + partial[r + 2].astype(F32)).astype(BF16)
    out_ref[...] = rs_in[N_DEV - 2].astype(F32) + partial[0].astype(F32)


def kernel(x, Wq, Wo, K_ext, V_ext):
    i = lax.axis_index("i")
    xb = x[0].astype(BF16)
    wq = Wq.astype(BF16)
    wo = Wo.astype(BF16)
    k = lax.dynamic_slice_in_dim(K_ext[0], i * HQ, HQ, axis=1)
    v = lax.dynamic_slice_in_dim(V_ext[0], i * HQ, HQ, axis=1)
    k = (jnp.transpose(k, (1, 0, 2)) * SCALE).astype(BF16)
    v = jnp.transpose(v, (1, 0, 2)).astype(BF16)

    out = pl.pallas_call(
        _body,
        out_shape=jax.ShapeDtypeStruct((SQ, D), F32),
        in_specs=[pl.BlockSpec(memory_space=pltpu.VMEM)] * 5,
        out_specs=pl.BlockSpec(memory_space=pltpu.VMEM),
        scratch_shapes=[
            pltpu.VMEM((N_DEV - 1, SQ, D), BF16),
            pltpu.VMEM((N_DEV, SQ, D), BF16),
            pltpu.VMEM((N_DEV - 1, SQ, D), BF16),
            pltpu.VMEM((SQ, D), BF16),
            pltpu.SemaphoreType.DMA((N_DEV - 1,)),
            pltpu.SemaphoreType.DMA((N_DEV - 1,)),
            pltpu.SemaphoreType.DMA((N_DEV - 1,)),
            pltpu.SemaphoreType.DMA((N_DEV - 1,)),
        ],
        compiler_params=pltpu.CompilerParams(collective_id=0),
    )(xb, wq, wo, k, v)
    return out[None]


# baseline (device time: 333390 ns/iter reference)
import jax
import jax.numpy as jnp
from jax import lax
from jax.experimental import pallas as pl
from jax.experimental.pallas import tpu as pltpu

N_DEV = 8
SQ = 512
D = 1024
HQ = 8
DH = 128
SKV = 2048
SCALE = 0.08838834764831843
BF16 = jnp.bfloat16
F32 = jnp.float32


def _attend(xb, wq_ref, wo_ref, k_ref, v_ref, qbuf, obuf):
    qbuf[...] = jnp.dot(xb, wq_ref[...], preferred_element_type=F32).astype(BF16)

    def head(h, carry):
        qh = qbuf[:, pl.ds(h * DH, DH)]
        kh = jnp.squeeze(k_ref[pl.ds(h, 1)], axis=0)
        vh = jnp.squeeze(v_ref[pl.ds(h, 1)], axis=0)
        s = lax.dot_general(qh, kh, (((1,), (1,)), ((), ())),
                            preferred_element_type=F32)
        m = jnp.max(s, axis=1, keepdims=True)
        p = jnp.exp(s - m)
        l = jnp.sum(p, axis=1, keepdims=True)
        o = jnp.dot(p.astype(BF16), vh, preferred_element_type=F32)
        obuf[:, pl.ds(h * DH, DH)] = (o / l).astype(BF16)
        return carry

    lax.fori_loop(0, HQ, head, 0)
    return jnp.dot(obuf[...], wo_ref[...], preferred_element_type=F32)


def _body(x_ref, wq_ref, wo_ref, k_ref, v_ref, out_ref,
          xfull, partial, rs_in, rs_out, qbuf, obuf,
          ag_send, ag_recv, rs_send, rs_recv):
    j = lax.axis_index("i")
    right = lax.rem(j + 1, N_DEV)
    left = lax.rem(j + N_DEV - 1, N_DEV)

    barrier = pltpu.get_barrier_semaphore()
    for nbr in (left, right):
        pl.semaphore_signal(barrier, inc=1, device_id=(nbr,),
                            device_id_type=pl.DeviceIdType.MESH)
    pl.semaphore_wait(barrier, 2)

    for t in range(N_DEV - 1):
        src = x_ref if t == 0 else xfull.at[t - 1]
        rdma = pltpu.make_async_remote_copy(
            src_ref=src,
            dst_ref=xfull.at[t],
            send_sem=ag_send.at[t],
            recv_sem=ag_recv.at[t],
            device_id=(right,),
            device_id_type=pl.DeviceIdType.MESH,
        )
        rdma.start()
        xb = x_ref[...] if t == 0 else xfull[t - 1]
        partial[t, :, :] = _attend(
            xb, wq_ref, wo_ref, k_ref, v_ref, qbuf, obuf).astype(BF16)
        rdma.wait()
    partial[N_DEV - 1, :, :] = _attend(
        xfull[N_DEV - 2], wq_ref, wo_ref, k_ref, v_ref, qbuf, obuf).astype(BF16)

    rs_out[...] = partial[1]
    for r in range(N_DEV - 1):
        rdma = pltpu.make_async_remote_copy(
            src_ref=rs_out,
            dst_ref=rs_in.at[r],
            send_sem=rs_send.at[r],
            recv_sem=rs_recv.at[r],
            device_id=(right,),
            device_id_type=pl.DeviceIdType.MESH,
        )
        rdma.start()
        rdma.wait()
        if r < N_DEV - 2:
            rs_out[...] = (rs_in[r].astype(F32)
                           + partial[r + 2].astype(F32)).astype(BF16)
    out_ref[...] = rs_in[N_DEV - 2].astype(F32) + partial[0].astype(F32)


def kernel(x, Wq, Wo, K_ext, V_ext):
    i = lax.axis_index("i")
    xb = x[0].astype(BF16)
    wq = Wq.astype(BF16)
    wo = Wo.astype(BF16)
    k = lax.dynamic_slice_in_dim(K_ext[0], i * HQ, HQ, axis=1)
    v = lax.dynamic_slice_in_dim(V_ext[0], i * HQ, HQ, axis=1)
    k = (jnp.transpose(k, (1, 0, 2)) * SCALE).astype(BF16)
    v = jnp.transpose(v, (1, 0, 2)).astype(BF16)

    out = pl.pallas_call(
        _body,
        out_shape=jax.ShapeDtypeStruct((SQ, D), F32),
        in_specs=[pl.BlockSpec(memory_space=pltpu.VMEM)] * 5,
        out_specs=pl.BlockSpec(memory_space=pltpu.VMEM),
        scratch_shapes=[
            pltpu.VMEM((N_DEV - 1, SQ, D), BF16),
            pltpu.VMEM((N_DEV, SQ, D), BF16),
            pltpu.VMEM((N_DEV - 1, SQ, D), BF16),
            pltpu.VMEM((SQ, D), BF16),
            pltpu.VMEM((SQ, D), BF16),
            pltpu.VMEM((SQ, D), BF16),
            pltpu.SemaphoreType.DMA((N_DEV - 1,)),
            pltpu.SemaphoreType.DMA((N_DEV - 1,)),
            pltpu.SemaphoreType.DMA((N_DEV - 1,)),
            pltpu.SemaphoreType.DMA((N_DEV - 1,)),
        ],
        compiler_params=pltpu.CompilerParams(
            collective_id=0, vmem_limit_bytes=100 * 1024 * 1024),
    )(xb, wq, wo, k, v)
    return out[None]


# device time: 253982 ns/iter; 1.3127x vs baseline; 1.3127x over previous
import jax
import jax.numpy as jnp
from jax import lax
from jax.experimental import pallas as pl
from jax.experimental.pallas import tpu as pltpu

N_DEV = 8
SQ = 512
D = 1024
HQ = 8
DH = 128
SKV = 2048
SCALE = 0.08838834764831843
BF16 = jnp.bfloat16
F32 = jnp.float32


def _attend(xb, wq_ref, wo_ref, k_ref, v_ref, qbuf, obuf):
    qbuf[...] = jnp.dot(xb, wq_ref[...], preferred_element_type=F32).astype(BF16)

    def head(h, carry):
        c = pl.ds(h * DH, DH)
        qh = qbuf[:, c]
        s = lax.dot_general(qh, k_ref[:, c], (((1,), (1,)), ((), ())),
                            preferred_element_type=F32)
        m = jnp.max(s, axis=1, keepdims=True)
        p = jnp.exp(s - m)
        l = jnp.sum(p, axis=1, keepdims=True)
        o = jnp.dot(p.astype(BF16), v_ref[:, c], preferred_element_type=F32)
        obuf[:, c] = (o / l).astype(BF16)
        return carry

    lax.fori_loop(0, HQ, head, 0)
    return jnp.dot(obuf[...], wo_ref[...], preferred_element_type=F32)


def _body(x_ref, wq_ref, wo_ref, k_ref, v_ref, out_ref,
          xfull, partial, rs_in, rs_out2, qbuf, obuf,
          ag_send, ag_recv, rs_send, rs_recv):
    j = lax.axis_index("i")
    right = lax.rem(j + 1, N_DEV)
    left = lax.rem(j + N_DEV - 1, N_DEV)

    barrier = pltpu.get_barrier_semaphore()
    for nbr in (left, right):
        pl.semaphore_signal(barrier, inc=1, device_id=(nbr,),
                            device_id_type=pl.DeviceIdType.MESH)
    pl.semaphore_wait(barrier, 2)

    def ag_rdma(t):
        return pltpu.make_async_remote_copy(
            src_ref=x_ref if t == 0 else xfull.at[t - 1],
            dst_ref=xfull.at[t],
            send_sem=ag_send.at[t],
            recv_sem=ag_recv.at[t],
            device_id=(right,),
            device_id_type=pl.DeviceIdType.MESH,
        )

    def rs_rdma(r):
        return pltpu.make_async_remote_copy(
            src_ref=partial.at[1] if r == 0 else rs_out2.at[r % 2],
            dst_ref=rs_in.at[r],
            send_sem=rs_send.at[r],
            recv_sem=rs_recv.at[r],
            device_id=(right,),
            device_id_type=pl.DeviceIdType.MESH,
        )

    def compute(t):
        xb = x_ref[...] if t == 0 else xfull[t - 1]
        partial[t, :, :] = _attend(
            xb, wq_ref, wo_ref, k_ref, v_ref, qbuf, obuf).astype(BF16)

    ag0 = ag_rdma(0)
    ag0.start()
    compute(0)
    ag0.wait()

    for t in range(1, N_DEV):
        ag = ag_rdma(t) if t < N_DEV - 1 else None
        if ag is not None:
            ag.start()
        compute(t)
        r = t - 1
        if r > 0:
            rs_rdma(r - 1).wait_recv()
            if r >= 2:
                rs_rdma(r - 2).wait_send()
            rs_out2[r % 2, :, :] = (rs_in[r - 1].astype(F32)
                                    + partial[r + 1].astype(F32)).astype(BF16)
        rs_rdma(r).start()
        if ag is not None:
            ag.wait()

    rs_rdma(N_DEV - 2).wait_recv()
    out_ref[...] = rs_in[N_DEV - 2].astype(F32) + partial[0].astype(F32)
    rs_rdma(N_DEV - 3).wait_send()
    rs_rdma(N_DEV - 2).wait_send()


def kernel(x, Wq, Wo, K_ext, V_ext):
    i = lax.axis_index("i")
    xb = x[0].astype(BF16)
    wq = Wq.astype(BF16)
    wo = Wo.astype(BF16)
    k = lax.dynamic_slice_in_dim(K_ext[0], i * HQ, HQ, axis=1)
    v = lax.dynamic_slice_in_dim(V_ext[0], i * HQ, HQ, axis=1)
    k = (k * SCALE).astype(BF16).reshape(SKV, HQ * DH)
    v = v.astype(BF16).reshape(SKV, HQ * DH)

    out = pl.pallas_call(
        _body,
        out_shape=jax.ShapeDtypeStruct((SQ, D), F32),
        in_specs=[pl.BlockSpec(memory_space=pltpu.VMEM)] * 5,
        out_specs=pl.BlockSpec(memory_space=pltpu.VMEM),
        scratch_shapes=[
            pltpu.VMEM((N_DEV - 1, SQ, D), BF16),
            pltpu.VMEM((N_DEV, SQ, D), BF16),
            pltpu.VMEM((N_DEV - 1, SQ, D), BF16),
            pltpu.VMEM((2, SQ, D), BF16),
            pltpu.VMEM((SQ, D), BF16),
            pltpu.VMEM((SQ, D), BF16),
            pltpu.SemaphoreType.DMA((N_DEV - 1,)),
            pltpu.SemaphoreType.DMA((N_DEV - 1,)),
            pltpu.SemaphoreType.DMA((N_DEV - 1,)),
            pltpu.SemaphoreType.DMA((N_DEV - 1,)),
        ],
        compiler_params=pltpu.CompilerParams(
            collective_id=0, vmem_limit_bytes=100 * 1024 * 1024),
    )(xb, wq, wo, k, v)
    return out[None]


# device time: 211473 ns/iter; 1.5765x vs baseline; 1.2010x over previous
import jax
import jax.numpy as jnp
from jax import lax
from jax.experimental import pallas as pl
from jax.experimental.pallas import tpu as pltpu

N_DEV = 8
SQ = 512
D = 1024
HQ = 8
DH = 128
SKV = 2048
SCALE = 0.08838834764831843
BF16 = jnp.bfloat16
F32 = jnp.float32


def _attend(xb, wq_ref, wo_ref, k_ref, v_ref, qbuf, obuf):
    qbuf[...] = jnp.dot(xb, wq_ref[...], preferred_element_type=F32).astype(BF16)

    def head(h, carry):
        c = pl.ds(h * DH, DH)
        qh = qbuf[:, c]
        s = jnp.dot(qh, k_ref[pl.ds(h * DH, DH), :],
                    preferred_element_type=F32)
        p = jnp.exp(s)
        l = jnp.sum(p, axis=1, keepdims=True)
        o = jnp.dot(p.astype(BF16), v_ref[:, c], preferred_element_type=F32)
        obuf[:, c] = (o * (1.0 / l)).astype(BF16)
        return carry

    lax.fori_loop(0, HQ, head, 0)
    return jnp.dot(obuf[...], wo_ref[...], preferred_element_type=F32)


def _body(x_ref, wq_ref, wo_ref, k_ref, v_ref, out_ref,
          xfull, partial, rs_in, rs_out2, qbuf, obuf,
          ag_send, ag_recv, rs_send, rs_recv):
    j = lax.axis_index("i")
    right = lax.rem(j + 1, N_DEV)
    left = lax.rem(j + N_DEV - 1, N_DEV)

    barrier = pltpu.get_barrier_semaphore()
    for nbr in (left, right):
        pl.semaphore_signal(barrier, inc=1, device_id=(nbr,),
                            device_id_type=pl.DeviceIdType.MESH)
    pl.semaphore_wait(barrier, 2)

    def ag_rdma(t):
        return pltpu.make_async_remote_copy(
            src_ref=x_ref if t == 0 else xfull.at[t - 1],
            dst_ref=xfull.at[t],
            send_sem=ag_send.at[t],
            recv_sem=ag_recv.at[t],
            device_id=(right,),
            device_id_type=pl.DeviceIdType.MESH,
        )

    def rs_rdma(r):
        return pltpu.make_async_remote_copy(
            src_ref=partial.at[1] if r == 0 else rs_out2.at[r % 2],
            dst_ref=rs_in.at[r],
            send_sem=rs_send.at[r],
            recv_sem=rs_recv.at[r],
            device_id=(right,),
            device_id_type=pl.DeviceIdType.MESH,
        )

    def compute(t):
        xb = x_ref[...] if t == 0 else xfull[t - 1]
        partial[t, :, :] = _attend(
            xb, wq_ref, wo_ref, k_ref, v_ref, qbuf, obuf).astype(BF16)

    ag0 = ag_rdma(0)
    ag0.start()
    compute(0)
    ag0.wait()

    for t in range(1, N_DEV):
        ag = ag_rdma(t) if t < N_DEV - 1 else None
        if ag is not None:
            ag.start()
        compute(t)
        r = t - 1
        if r > 0:
            rs_rdma(r - 1).wait_recv()
            if r >= 2:
                rs_rdma(r - 2).wait_send()
            rs_out2[r % 2, :, :] = (rs_in[r - 1].astype(F32)
                                    + partial[r + 1].astype(F32)).astype(BF16)
        rs_rdma(r).start()
        if ag is not None:
            ag.wait()

    rs_rdma(N_DEV - 2).wait_recv()
    out_ref[...] = rs_in[N_DEV - 2].astype(F32) + partial[0].astype(F32)
    rs_rdma(N_DEV - 3).wait_send()
    rs_rdma(N_DEV - 2).wait_send()


def kernel(x, Wq, Wo, K_ext, V_ext):
    i = lax.axis_index("i")
    xb = x[0].astype(BF16)
    wq = Wq.astype(BF16)
    wo = Wo.astype(BF16)
    k = lax.dynamic_slice_in_dim(K_ext[0], i * HQ, HQ, axis=1)
    v = lax.dynamic_slice_in_dim(V_ext[0], i * HQ, HQ, axis=1)
    k = (k * SCALE).astype(BF16)
    k = jnp.transpose(k, (1, 2, 0)).reshape(HQ * DH, SKV)
    v = v.astype(BF16).reshape(SKV, HQ * DH)

    out = pl.pallas_call(
        _body,
        out_shape=jax.ShapeDtypeStruct((SQ, D), F32),
        in_specs=[pl.BlockSpec(memory_space=pltpu.VMEM)] * 5,
        out_specs=pl.BlockSpec(memory_space=pltpu.VMEM),
        scratch_shapes=[
            pltpu.VMEM((N_DEV - 1, SQ, D), BF16),
            pltpu.VMEM((N_DEV, SQ, D), BF16),
            pltpu.VMEM((N_DEV - 1, SQ, D), BF16),
            pltpu.VMEM((2, SQ, D), BF16),
            pltpu.VMEM((SQ, D), BF16),
            pltpu.VMEM((SQ, D), BF16),
            pltpu.SemaphoreType.DMA((N_DEV - 1,)),
            pltpu.SemaphoreType.DMA((N_DEV - 1,)),
            pltpu.SemaphoreType.DMA((N_DEV - 1,)),
            pltpu.SemaphoreType.DMA((N_DEV - 1,)),
        ],
        compiler_params=pltpu.CompilerParams(
            collective_id=0, vmem_limit_bytes=100 * 1024 * 1024),
    )(xb, wq, wo, k, v)
    return out[None]


# device time: 207782 ns/iter; 1.6045x vs baseline; 1.0178x over previous
import jax
import jax.numpy as jnp
from jax import lax
from jax.experimental import pallas as pl
from jax.experimental.pallas import tpu as pltpu

N_DEV = 8
SQ = 512
D = 1024
HQ = 8
DH = 128
SKV = 2048
SCALE = 0.08838834764831843
BF16 = jnp.bfloat16
F32 = jnp.float32


def _attend(xb, wq_ref, wo_ref, k_ref, v_ref, qbuf, obuf):
    qbuf[...] = jnp.dot(xb, wq_ref[...], preferred_element_type=F32).astype(BF16)

    for h in range(HQ):
        c = slice(h * DH, (h + 1) * DH)
        s = jnp.dot(qbuf[:, c], k_ref[c, :],
                    preferred_element_type=F32)
        p = jnp.exp(s)
        l = jnp.sum(p, axis=1, keepdims=True)
        o = jnp.dot(p.astype(BF16), v_ref[:, c], preferred_element_type=F32)
        obuf[:, c] = (o * (1.0 / l)).astype(BF16)
    return jnp.dot(obuf[...], wo_ref[...], preferred_element_type=F32)


def _body(x_ref, wq_ref, wo_ref, k_ref, v_ref, out_ref,
          xfull, partial, rs_in, rs_out2, qbuf, obuf,
          ag_send, ag_recv, rs_send, rs_recv):
    j = lax.axis_index("i")
    right = lax.rem(j + 1, N_DEV)
    left = lax.rem(j + N_DEV - 1, N_DEV)

    barrier = pltpu.get_barrier_semaphore()
    for nbr in (left, right):
        pl.semaphore_signal(barrier, inc=1, device_id=(nbr,),
                            device_id_type=pl.DeviceIdType.MESH)
    pl.semaphore_wait(barrier, 2)

    def ag_rdma(t):
        return pltpu.make_async_remote_copy(
            src_ref=x_ref if t == 0 else xfull.at[t - 1],
            dst_ref=xfull.at[t],
            send_sem=ag_send.at[t],
            recv_sem=ag_recv.at[t],
            device_id=(right,),
            device_id_type=pl.DeviceIdType.MESH,
        )

    def rs_rdma(r):
        return pltpu.make_async_remote_copy(
            src_ref=partial.at[1] if r == 0 else rs_out2.at[r % 2],
            dst_ref=rs_in.at[r],
            send_sem=rs_send.at[r],
            recv_sem=rs_recv.at[r],
            device_id=(right,),
            device_id_type=pl.DeviceIdType.MESH,
        )

    def compute(t):
        xb = x_ref[...] if t == 0 else xfull[t - 1]
        partial[t, :, :] = _attend(
            xb, wq_ref, wo_ref, k_ref, v_ref, qbuf, obuf).astype(BF16)

    ag0 = ag_rdma(0)
    ag0.start()
    compute(0)
    ag0.wait()

    for t in range(1, N_DEV):
        ag = ag_rdma(t) if t < N_DEV - 1 else None
        if ag is not None:
            ag.start()
        compute(t)
        r = t - 1
        if r > 0:
            rs_rdma(r - 1).wait_recv()
            if r >= 2:
                rs_rdma(r - 2).wait_send()
            rs_out2[r % 2, :, :] = (rs_in[r - 1].astype(F32)
                                    + partial[r + 1].astype(F32)).astype(BF16)
        rs_rdma(r).start()
        if ag is not None:
            ag.wait()

    rs_rdma(N_DEV - 2).wait_recv()
    out_ref[...] = rs_in[N_DEV - 2].astype(F32) + partial[0].astype(F32)
    rs_rdma(N_DEV - 3).wait_send()
    rs_rdma(N_DEV - 2).wait_send()


def kernel(x, Wq, Wo, K_ext, V_ext):
    i = lax.axis_index("i")
    xb = x[0].astype(BF16)
    wq = Wq.astype(BF16)
    wo = Wo.astype(BF16)
    k = lax.dynamic_slice_in_dim(K_ext[0], i * HQ, HQ, axis=1)
    v = lax.dynamic_slice_in_dim(V_ext[0], i * HQ, HQ, axis=1)
    k = (k * SCALE).astype(BF16)
    k = jnp.transpose(k, (1, 2, 0)).reshape(HQ * DH, SKV)
    v = v.astype(BF16).reshape(SKV, HQ * DH)

    out = pl.pallas_call(
        _body,
        out_shape=jax.ShapeDtypeStruct((SQ, D), F32),
        in_specs=[pl.BlockSpec(memory_space=pltpu.VMEM)] * 5,
        out_specs=pl.BlockSpec(memory_space=pltpu.VMEM),
        scratch_shapes=[
            pltpu.VMEM((N_DEV - 1, SQ, D), BF16),
            pltpu.VMEM((N_DEV, SQ, D), BF16),
            pltpu.VMEM((N_DEV - 1, SQ, D), BF16),
            pltpu.VMEM((2, SQ, D), BF16),
            pltpu.VMEM((SQ, D), BF16),
            pltpu.VMEM((SQ, D), BF16),
            pltpu.SemaphoreType.DMA((N_DEV - 1,)),
            pltpu.SemaphoreType.DMA((N_DEV - 1,)),
            pltpu.SemaphoreType.DMA((N_DEV - 1,)),
            pltpu.SemaphoreType.DMA((N_DEV - 1,)),
        ],
        compiler_params=pltpu.CompilerParams(
            collective_id=0, vmem_limit_bytes=100 * 1024 * 1024),
    )(xb, wq, wo, k, v)
    return out[None]


# device time: 163917 ns/iter; 2.0339x vs baseline; 1.2676x over previous
import jax
import jax.numpy as jnp
from jax import lax
from jax.experimental import pallas as pl
from jax.experimental.pallas import tpu as pltpu

N_DEV = 8
SQ = 512
HSQ = SQ // 2
D = 1024
HQ = 8
DH = 128
SKV = 2048
SCALE = 0.08838834764831843
BF16 = jnp.bfloat16
F32 = jnp.float32


def _attend(xb, wq_ref, wo_ref, k_ref, v_ref, qbuf, obuf):
    qbuf[...] = jnp.dot(xb, wq_ref[...], preferred_element_type=F32).astype(BF16)

    def head(h, carry):
        c = pl.ds(h * DH, DH)
        s = jnp.dot(qbuf[:, c], k_ref[c, :],
                    preferred_element_type=F32)
        p = jnp.exp(s)
        l = jnp.sum(p, axis=1, keepdims=True)
        o = jnp.dot(p.astype(BF16), v_ref[:, c], preferred_element_type=F32)
        obuf[:, c] = (o * (1.0 / l)).astype(BF16)
        return carry

    lax.fori_loop(0, HQ, head, 0)
    return jnp.dot(obuf[...], wo_ref[...], preferred_element_type=F32)


def _body(x_ref, wq_ref, wo_ref, k_ref, v_ref, out_ref,
          xfull, partial, rs_in, rs_out2, qbuf, obuf,
          ag_send, ag_recv, rs_send, rs_recv):
    j = lax.axis_index("i")
    right = lax.rem(j + 1, N_DEV)
    left = lax.rem(j + N_DEV - 1, N_DEV)

    barrier = pltpu.get_barrier_semaphore()
    for nbr in (left, right):
        pl.semaphore_signal(barrier, inc=1, device_id=(nbr,),
                            device_id_type=pl.DeviceIdType.MESH)
    pl.semaphore_wait(barrier, 2)

    def _rows(d):
        return pl.ds(d * HSQ, HSQ)

    def ag_rdma(t, d):
        src = (x_ref.at[_rows(d), :] if t == 0
               else xfull.at[t - 1, _rows(d), :])
        return pltpu.make_async_remote_copy(
            src_ref=src,
            dst_ref=xfull.at[t, _rows(d), :],
            send_sem=ag_send.at[t, d],
            recv_sem=ag_recv.at[t, d],
            device_id=(right if d == 0 else left,),
            device_id_type=pl.DeviceIdType.MESH,
        )

    def rs_rdma(r, d):
        src = (partial.at[1, _rows(d), :] if r == 0
               else rs_out2.at[r % 2, _rows(d), :])
        return pltpu.make_async_remote_copy(
            src_ref=src,
            dst_ref=rs_in.at[r, _rows(d), :],
            send_sem=rs_send.at[r, d],
            recv_sem=rs_recv.at[r, d],
            device_id=(right if d == 0 else left,),
            device_id_type=pl.DeviceIdType.MESH,
        )

    def compute(t):
        xb = x_ref[...] if t == 0 else xfull[t - 1]
        partial[t, :, :] = _attend(
            xb, wq_ref, wo_ref, k_ref, v_ref, qbuf, obuf).astype(BF16)

    ag0 = (ag_rdma(0, 0), ag_rdma(0, 1))
    ag0[0].start()
    ag0[1].start()
    compute(0)
    ag0[0].wait()
    ag0[1].wait()

    for t in range(1, N_DEV):
        ag = (ag_rdma(t, 0), ag_rdma(t, 1)) if t < N_DEV - 1 else None
        if ag is not None:
            ag[0].start()
            ag[1].start()
        compute(t)
        r = t - 1
        if r > 0:
            rs_rdma(r - 1, 0).wait_recv()
            rs_rdma(r - 1, 1).wait_recv()
            if r >= 2:
                rs_rdma(r - 2, 0).wait_send()
                rs_rdma(r - 2, 1).wait_send()
            rs_out2[r % 2, :, :] = (rs_in[r - 1].astype(F32)
                                    + partial[r + 1].astype(F32)).astype(BF16)
        rs_rdma(r, 0).start()
        rs_rdma(r, 1).start()
        if ag is not None:
            ag[0].wait()
            ag[1].wait()

    rs_rdma(N_DEV - 2, 0).wait_recv()
    rs_rdma(N_DEV - 2, 1).wait_recv()
    out_ref[...] = rs_in[N_DEV - 2].astype(F32) + partial[0].astype(F32)
    for d in (0, 1):
        rs_rdma(N_DEV - 3, d).wait_send()
        rs_rdma(N_DEV - 2, d).wait_send()


def kernel(x, Wq, Wo, K_ext, V_ext):
    i = lax.axis_index("i")
    xb = x[0].astype(BF16)
    wq = Wq.astype(BF16)
    wo = Wo.astype(BF16)
    k = lax.dynamic_slice_in_dim(K_ext[0], i * HQ, HQ, axis=1)
    v = lax.dynamic_slice_in_dim(V_ext[0], i * HQ, HQ, axis=1)
    k = (k * SCALE).astype(BF16)
    k = jnp.transpose(k, (1, 2, 0)).reshape(HQ * DH, SKV)
    v = v.astype(BF16).reshape(SKV, HQ * DH)

    out = pl.pallas_call(
        _body,
        out_shape=jax.ShapeDtypeStruct((SQ, D), F32),
        in_specs=[pl.BlockSpec(memory_space=pltpu.VMEM)] * 5,
        out_specs=pl.BlockSpec(memory_space=pltpu.VMEM),
        scratch_shapes=[
            pltpu.VMEM((N_DEV - 1, SQ, D), BF16),
            pltpu.VMEM((N_DEV, SQ, D), BF16),
            pltpu.VMEM((N_DEV - 1, SQ, D), BF16),
            pltpu.VMEM((2, SQ, D), BF16),
            pltpu.VMEM((SQ, D), BF16),
            pltpu.VMEM((SQ, D), BF16),
            pltpu.SemaphoreType.DMA((N_DEV - 1, 2)),
            pltpu.SemaphoreType.DMA((N_DEV - 1, 2)),
            pltpu.SemaphoreType.DMA((N_DEV - 1, 2)),
            pltpu.SemaphoreType.DMA((N_DEV - 1, 2)),
        ],
        compiler_params=pltpu.CompilerParams(
            collective_id=0, vmem_limit_bytes=100 * 1024 * 1024),
    )(xb, wq, wo, k, v)
    return out[None]


# device time: 154543 ns/iter; 2.1573x vs baseline; 1.0607x over previous
import jax
import jax.numpy as jnp
from jax import lax
from jax.experimental import pallas as pl
from jax.experimental.pallas import tpu as pltpu

N_DEV = 8
SQ = 512
HSQ = SQ // 2
D = 1024
HQ = 8
DH = 128
SKV = 2048
SCALE = 0.08838834764831843
BF16 = jnp.bfloat16
F32 = jnp.float32


def _attend(xb, wq_ref, wo_ref, k_ref, v_ref, qbuf, obuf):
    qbuf[...] = jnp.dot(xb, wq_ref[...], preferred_element_type=F32).astype(BF16)

    def head_pair(hh, carry):
        c0 = pl.ds(2 * hh * DH, DH)
        c1 = pl.ds((2 * hh + 1) * DH, DH)
        s0 = jnp.dot(qbuf[:, c0], k_ref[c0, :], preferred_element_type=F32)
        s1 = jnp.dot(qbuf[:, c1], k_ref[c1, :], preferred_element_type=F32)
        p0 = jnp.exp(s0)
        p1 = jnp.exp(s1)
        l0 = jnp.sum(p0, axis=1, keepdims=True)
        l1 = jnp.sum(p1, axis=1, keepdims=True)
        o0 = jnp.dot(p0.astype(BF16), v_ref[:, c0], preferred_element_type=F32)
        o1 = jnp.dot(p1.astype(BF16), v_ref[:, c1], preferred_element_type=F32)
        obuf[:, c0] = (o0 * (1.0 / l0)).astype(BF16)
        obuf[:, c1] = (o1 * (1.0 / l1)).astype(BF16)
        return carry

    lax.fori_loop(0, HQ // 2, head_pair, 0)
    return jnp.dot(obuf[...], wo_ref[...], preferred_element_type=F32)


def _body(x_ref, wq_ref, wo_ref, k_ref, v_ref, out_ref,
          xfull, partial, rs_in, rs_out2, qbuf, obuf,
          ag_send, ag_recv, rs_send, rs_recv):
    j = lax.axis_index("i")
    right = lax.rem(j + 1, N_DEV)
    left = lax.rem(j + N_DEV - 1, N_DEV)

    barrier = pltpu.get_barrier_semaphore()
    for nbr in (left, right):
        pl.semaphore_signal(barrier, inc=1, device_id=(nbr,),
                            device_id_type=pl.DeviceIdType.MESH)
    pl.semaphore_wait(barrier, 2)

    def _rows(d):
        return pl.ds(d * HSQ, HSQ)

    def ag_rdma(t, d):
        src = (x_ref.at[_rows(d), :] if t == 0
               else xfull.at[t - 1, _rows(d), :])
        return pltpu.make_async_remote_copy(
            src_ref=src,
            dst_ref=xfull.at[t, _rows(d), :],
            send_sem=ag_send.at[t, d],
            recv_sem=ag_recv.at[t, d],
            device_id=(right if d == 0 else left,),
            device_id_type=pl.DeviceIdType.MESH,
        )

    def rs_rdma(r, d):
        src = (partial.at[1, _rows(d), :] if r == 0
               else rs_out2.at[r % 2, _rows(d), :])
        return pltpu.make_async_remote_copy(
            src_ref=src,
            dst_ref=rs_in.at[r, _rows(d), :],
            send_sem=rs_send.at[r, d],
            recv_sem=rs_recv.at[r, d],
            device_id=(right if d == 0 else left,),
            device_id_type=pl.DeviceIdType.MESH,
        )

    def compute(t):
        xb = x_ref[...] if t == 0 else xfull[t - 1]
        partial[t, :, :] = _attend(
            xb, wq_ref, wo_ref, k_ref, v_ref, qbuf, obuf).astype(BF16)

    ag0 = (ag_rdma(0, 0), ag_rdma(0, 1))
    ag0[0].start()
    ag0[1].start()
    compute(0)
    ag0[0].wait()
    ag0[1].wait()

    for t in range(1, N_DEV):
        ag = (ag_rdma(t, 0), ag_rdma(t, 1)) if t < N_DEV - 1 else None
        if ag is not None:
            ag[0].start()
            ag[1].start()
        compute(t)
        r = t - 1
        if r > 0:
            rs_rdma(r - 1, 0).wait_recv()
            rs_rdma(r - 1, 1).wait_recv()
            if r >= 2:
                rs_rdma(r - 2, 0).wait_send()
                rs_rdma(r - 2, 1).wait_send()
            rs_out2[r % 2, :, :] = (rs_in[r - 1].astype(F32)
                                    + partial[r + 1].astype(F32)).astype(BF16)
        rs_rdma(r, 0).start()
        rs_rdma(r, 1).start()
        if ag is not None:
            ag[0].wait()
            ag[1].wait()

    rs_rdma(N_DEV - 2, 0).wait_recv()
    rs_rdma(N_DEV - 2, 1).wait_recv()
    out_ref[...] = rs_in[N_DEV - 2].astype(F32) + partial[0].astype(F32)
    for d in (0, 1):
        rs_rdma(N_DEV - 3, d).wait_send()
        rs_rdma(N_DEV - 2, d).wait_send()


def kernel(x, Wq, Wo, K_ext, V_ext):
    i = lax.axis_index("i")
    xb = x[0].astype(BF16)
    wq = Wq.astype(BF16)
    wo = Wo.astype(BF16)
    k = lax.dynamic_slice_in_dim(K_ext[0], i * HQ, HQ, axis=1)
    v = lax.dynamic_slice_in_dim(V_ext[0], i * HQ, HQ, axis=1)
    k = (k * SCALE).astype(BF16)
    k = jnp.transpose(k, (1, 2, 0)).reshape(HQ * DH, SKV)
    v = v.astype(BF16).reshape(SKV, HQ * DH)

    out = pl.pallas_call(
        _body,
        out_shape=jax.ShapeDtypeStruct((SQ, D), F32),
        in_specs=[pl.BlockSpec(memory_space=pltpu.VMEM)] * 5,
        out_specs=pl.BlockSpec(memory_space=pltpu.VMEM),
        scratch_shapes=[
            pltpu.VMEM((N_DEV - 1, SQ, D), BF16),
            pltpu.VMEM((N_DEV, SQ, D), BF16),
            pltpu.VMEM((N_DEV - 1, SQ, D), BF16),
            pltpu.VMEM((2, SQ, D), BF16),
            pltpu.VMEM((SQ, D), BF16),
            pltpu.VMEM((SQ, D), BF16),
            pltpu.SemaphoreType.DMA((N_DEV - 1, 2)),
            pltpu.SemaphoreType.DMA((N_DEV - 1, 2)),
            pltpu.SemaphoreType.DMA((N_DEV - 1, 2)),
            pltpu.SemaphoreType.DMA((N_DEV - 1, 2)),
        ],
        compiler_params=pltpu.CompilerParams(
            collective_id=0, vmem_limit_bytes=100 * 1024 * 1024),
    )(xb, wq, wo, k, v)
    return out[None]


# device time: 154411 ns/iter; 2.1591x vs baseline; 1.0009x over previous
import jax
import jax.numpy as jnp
from jax import lax
from jax.experimental import pallas as pl
from jax.experimental.pallas import tpu as pltpu

N_DEV = 8
SQ = 512
HSQ = SQ // 2
D = 1024
HQ = 8
DH = 128
SKV = 2048
SCALE = 0.08838834764831843
BF16 = jnp.bfloat16
F32 = jnp.float32


def _attend(xb, wq_ref, wo_ref, k_ref, v_ref, qbuf, obuf):
    qbuf[...] = jnp.dot(xb, wq_ref[...], preferred_element_type=F32).astype(BF16)

    def head_quad(hh, carry):
        cs = [pl.ds((4 * hh + u) * DH, DH) for u in range(4)]
        ss = [jnp.dot(qbuf[:, c], k_ref[c, :], preferred_element_type=F32)
              for c in cs]
        ps = [jnp.exp(sv) for sv in ss]
        ls = [jnp.sum(pv, axis=1, keepdims=True) for pv in ps]
        os_ = [jnp.dot(pv.astype(BF16), v_ref[:, c], preferred_element_type=F32)
               for pv, c in zip(ps, cs)]
        for c, ov, lv in zip(cs, os_, ls):
            obuf[:, c] = (ov * (1.0 / lv)).astype(BF16)
        return carry

    lax.fori_loop(0, HQ // 4, head_quad, 0)
    return jnp.dot(obuf[...], wo_ref[...], preferred_element_type=F32)


def _body(x_ref, wq_ref, wo_ref, k_ref, v_ref, out_ref,
          xfull, partial, rs_in, rs_out2, qbuf, obuf,
          ag_send, ag_recv, rs_send, rs_recv):
    j = lax.axis_index("i")
    right = lax.rem(j + 1, N_DEV)
    left = lax.rem(j + N_DEV - 1, N_DEV)

    barrier = pltpu.get_barrier_semaphore()
    for nbr in (left, right):
        pl.semaphore_signal(barrier, inc=1, device_id=(nbr,),
                            device_id_type=pl.DeviceIdType.MESH)
    pl.semaphore_wait(barrier, 2)

    def _rows(d):
        return pl.ds(d * HSQ, HSQ)

    def ag_rdma(t, d):
        src = (x_ref.at[_rows(d), :] if t == 0
               else xfull.at[t - 1, _rows(d), :])
        return pltpu.make_async_remote_copy(
            src_ref=src,
            dst_ref=xfull.at[t, _rows(d), :],
            send_sem=ag_send.at[t, d],
            recv_sem=ag_recv.at[t, d],
            device_id=(right if d == 0 else left,),
            device_id_type=pl.DeviceIdType.MESH,
        )

    def rs_rdma(r, d):
        src = (partial.at[1, _rows(d), :] if r == 0
               else rs_out2.at[r % 2, _rows(d), :])
        return pltpu.make_async_remote_copy(
            src_ref=src,
            dst_ref=rs_in.at[r, _rows(d), :],
            send_sem=rs_send.at[r, d],
            recv_sem=rs_recv.at[r, d],
            device_id=(right if d == 0 else left,),
            device_id_type=pl.DeviceIdType.MESH,
        )

    def attend_block(t):
        xb = x_ref[...] if t == 0 else xfull[t - 1]
        return _attend(xb, wq_ref, wo_ref, k_ref, v_ref, qbuf, obuf)

    ag0 = (ag_rdma(0, 0), ag_rdma(0, 1))
    ag0[0].start()
    ag0[1].start()
    partial[0, :, :] = attend_block(0).astype(BF16)
    ag0[0].wait()
    ag0[1].wait()

    for t in range(1, N_DEV):
        ag = (ag_rdma(t, 0), ag_rdma(t, 1)) if t < N_DEV - 1 else None
        if ag is not None:
            ag[0].start()
            ag[1].start()
        res = attend_block(t)
        r = t - 1
        if r == 0:
            partial[1, :, :] = res.astype(BF16)
        else:
            rs_rdma(r - 1, 0).wait_recv()
            rs_rdma(r - 1, 1).wait_recv()
            if r >= 2:
                rs_rdma(r - 2, 0).wait_send()
                rs_rdma(r - 2, 1).wait_send()
            rs_out2[r % 2, :, :] = (rs_in[r - 1].astype(F32) + res).astype(BF16)
        rs_rdma(r, 0).start()
        rs_rdma(r, 1).start()
        if ag is not None:
            ag[0].wait()
            ag[1].wait()

    rs_rdma(N_DEV - 2, 0).wait_recv()
    rs_rdma(N_DEV - 2, 1).wait_recv()
    out_ref[...] = rs_in[N_DEV - 2].astype(F32) + partial[0].astype(F32)
    for d in (0, 1):
        rs_rdma(N_DEV - 3, d).wait_send()
        rs_rdma(N_DEV - 2, d).wait_send()


def kernel(x, Wq, Wo, K_ext, V_ext):
    i = lax.axis_index("i")
    xb = x[0].astype(BF16)
    wq = Wq.astype(BF16)
    wo = Wo.astype(BF16)
    k = lax.dynamic_slice_in_dim(K_ext[0], i * HQ, HQ, axis=1)
    v = lax.dynamic_slice_in_dim(V_ext[0], i * HQ, HQ, axis=1)
    k = (k * SCALE).astype(BF16)
    k = jnp.transpose(k, (1, 2, 0)).reshape(HQ * DH, SKV)
    v = v.astype(BF16).reshape(SKV, HQ * DH)

    out = pl.pallas_call(
        _body,
        out_shape=jax.ShapeDtypeStruct((SQ, D), F32),
        in_specs=[pl.BlockSpec(memory_space=pltpu.VMEM)] * 5,
        out_specs=pl.BlockSpec(memory_space=pltpu.VMEM),
        scratch_shapes=[
            pltpu.VMEM((N_DEV - 1, SQ, D), BF16),
            pltpu.VMEM((2, SQ, D), BF16),
            pltpu.VMEM((N_DEV - 1, SQ, D), BF16),
            pltpu.VMEM((2, SQ, D), BF16),
            pltpu.VMEM((SQ, D), BF16),
            pltpu.VMEM((SQ, D), BF16),
            pltpu.SemaphoreType.DMA((N_DEV - 1, 2)),
            pltpu.SemaphoreType.DMA((N_DEV - 1, 2)),
            pltpu.SemaphoreType.DMA((N_DEV - 1, 2)),
            pltpu.SemaphoreType.DMA((N_DEV - 1, 2)),
        ],
        compiler_params=pltpu.CompilerParams(
            collective_id=0, vmem_limit_bytes=100 * 1024 * 1024),
    )(xb, wq, wo, k, v)
    return out[None]
